# baseline (speedup 1.0000x reference)
import jax
import jax.numpy as jnp
from jax.experimental import pallas as pl
from jax.experimental.pallas import tpu as pltpu

R = 1024
NH = 16
DH = 64
SCALE = 0.125


def _fused_body(xs_ref, xd_ref, wq_ref, wk_ref, wv_ref, wo_ref, out_ref,
                q_s, k_s, v_s, acc_s):
    c = pl.program_id(1)
    bf16 = jnp.bfloat16
    f32 = jnp.float32

    @pl.when(c != 1)
    def _():
        q_s[...] = (jnp.dot(xs_ref[0].astype(bf16),
                            wq_ref[...].astype(bf16),
                            preferred_element_type=f32) * SCALE).astype(bf16)

    @pl.when(c != 2)
    def _():
        xd = xd_ref[0].astype(bf16)
        k_s[...] = jnp.dot(xd, wk_ref[...].astype(bf16),
                           preferred_element_type=f32).astype(bf16)
        v_s[...] = jnp.dot(xd, wv_ref[...].astype(bf16),
                           preferred_element_type=f32).astype(bf16)

    coef = jnp.where(c == 0, 2.0 / 3.0, jnp.where(c == 1, 1.0 / 3.0, 1.0))
    for h in range(NH):
        sl = slice(h * DH, (h + 1) * DH)
        q = q_s[:, sl]
        k = k_s[:, sl]
        v = v_s[:, sl]
        qf = q.astype(f32)
        kf = k.astype(f32)
        qn = jnp.sum(qf * qf, axis=1)
        kn = jnp.sum(kf * kf, axis=1)
        m = jnp.sqrt(qn * jnp.max(kn))
        s = jax.lax.dot_general(q, k, (((1,), (1,)), ((), ())),
                                preferred_element_type=f32)
        ve = jnp.concatenate([v, jnp.ones((R, 1), bf16)], axis=1)
        p = jnp.exp((s - m[:, None]).astype(bf16))
        o_l = jnp.dot(p, ve, preferred_element_type=f32)
        o = o_l[:, :DH] * (coef / o_l[:, DH])[:, None]

        @pl.when(c == 1)
        def _():
            acc_s[:, sl] += o

        @pl.when(c != 1)
        def _():
            acc_s[:, sl] = o

    @pl.when(c >= 1)
    def _():
        out_ref[0] = jnp.dot(acc_s[...].astype(bf16),
                             wo_ref[...].astype(bf16),
                             preferred_element_type=f32)


def kernel(x, Wq, Wk, Wv, Wo):
    B, N, D = x.shape
    bf16 = jnp.bfloat16
    return pl.pallas_call(
        _fused_body,
        grid=(B, 3),
        in_specs=[
            pl.BlockSpec((1, R, D), lambda b, c: (b, c // 2, 0)),
            pl.BlockSpec((1, R, D), lambda b, c: (b, (c + 3) // 2, 0)),
            pl.BlockSpec((D, D), lambda b, c: (0, 0)),
            pl.BlockSpec((D, D), lambda b, c: (0, 0)),
            pl.BlockSpec((D, D), lambda b, c: (0, 0)),
            pl.BlockSpec((D, D), lambda b, c: (0, 0)),
        ],
        out_specs=pl.BlockSpec((1, R, D), lambda b, c: (b, c // 2, 0)),
        out_shape=jax.ShapeDtypeStruct((B, N, D), jnp.float32),
        scratch_shapes=[
            pltpu.VMEM((R, D), bf16),
            pltpu.VMEM((R, D), bf16),
            pltpu.VMEM((R, D), bf16),
            pltpu.VMEM((R, D), jnp.float32),
        ],
        input_output_aliases={0: 0},
        compiler_params=pltpu.CompilerParams(
            dimension_semantics=("parallel", "arbitrary")),
    )(x, x, Wq, Wk, Wv, Wo)


# speedup vs baseline: 1.0262x; 1.0262x over previous
import jax
import jax.numpy as jnp
from jax.experimental import pallas as pl
from jax.experimental.pallas import tpu as pltpu

R = 1024
NH = 16
DH = 64
SCALE = 0.125 * 1.4426950408889634


def _fused_body(xs_ref, xd_ref, wq_ref, wk_ref, wv_ref, wo_ref, out_ref,
                q_s, k_s, v_s, acc_s):
    c = pl.program_id(1)
    bf16 = jnp.bfloat16
    f32 = jnp.float32

    @pl.when(c != 1)
    def _():
        q_s[...] = (jnp.dot(xs_ref[0].astype(bf16),
                            wq_ref[...].astype(bf16),
                            preferred_element_type=f32) * SCALE).astype(bf16)

    @pl.when(c != 2)
    def _():
        xd = xd_ref[0].astype(bf16)
        k_s[...] = jnp.dot(xd, wk_ref[...].astype(bf16),
                           preferred_element_type=f32).astype(bf16)
        v_s[...] = jnp.dot(xd, wv_ref[...].astype(bf16),
                           preferred_element_type=f32).astype(bf16)

    coef = jnp.where(c == 0, 2.0 / 3.0, jnp.where(c == 1, 1.0 / 3.0, 1.0))
    for h in range(NH):
        sl = slice(h * DH, (h + 1) * DH)
        q = q_s[:, sl]
        k = k_s[:, sl]
        v = v_s[:, sl]
        qf = q.astype(f32)
        kf = k.astype(f32)
        qn = jnp.sum(qf * qf, axis=1)
        kn = jnp.sum(kf * kf, axis=1)
        m = jnp.sqrt(qn * jnp.max(kn))
        s = jax.lax.dot_general(q, k, (((1,), (1,)), ((), ())),
                                preferred_element_type=f32)
        p = jnp.exp2((s - m[:, None]).astype(bf16))
        l = jnp.sum(p.astype(f32), axis=1)
        o = jnp.dot(p, v, preferred_element_type=f32)
        o = o * (coef / l)[:, None]

        @pl.when(c == 1)
        def _():
            acc_s[:, sl] += o

        @pl.when(c != 1)
        def _():
            acc_s[:, sl] = o

    @pl.when(c >= 1)
    def _():
        out_ref[0] = jnp.dot(acc_s[...].astype(bf16),
                             wo_ref[...].astype(bf16),
                             preferred_element_type=f32)


def kernel(x, Wq, Wk, Wv, Wo):
    B, N, D = x.shape
    bf16 = jnp.bfloat16
    return pl.pallas_call(
        _fused_body,
        grid=(B, 3),
        in_specs=[
            pl.BlockSpec((1, R, D), lambda b, c: (b, c // 2, 0)),
            pl.BlockSpec((1, R, D), lambda b, c: (b, (c + 3) // 2, 0)),
            pl.BlockSpec((D, D), lambda b, c: (0, 0)),
            pl.BlockSpec((D, D), lambda b, c: (0, 0)),
            pl.BlockSpec((D, D), lambda b, c: (0, 0)),
            pl.BlockSpec((D, D), lambda b, c: (0, 0)),
        ],
        out_specs=pl.BlockSpec((1, R, D), lambda b, c: (b, c // 2, 0)),
        out_shape=jax.ShapeDtypeStruct((B, N, D), jnp.float32),
        scratch_shapes=[
            pltpu.VMEM((R, D), bf16),
            pltpu.VMEM((R, D), bf16),
            pltpu.VMEM((R, D), bf16),
            pltpu.VMEM((R, D), jnp.float32),
        ],
        input_output_aliases={0: 0},
        compiler_params=pltpu.CompilerParams(
            dimension_semantics=("parallel", "arbitrary")),
    )(x, x, Wq, Wk, Wv, Wo)
